# baseline (speedup 1.0000x reference)
import functools

import jax
import jax.numpy as jnp
from jax import lax
from jax.experimental import pallas as pl
from jax.experimental.pallas import tpu as pltpu
from jax.experimental.pallas import tpu_sc as plsc

B, F, V, D = 16384, 26, 100000, 16
CHUNK = 128
CF = CHUNK * F
NSLICE = CF // 16


def _fm_body(idx_hbm, coef_hbm, w0_hbm, first_hbm, second_hbm,
             score_hbm, emb_hbm,
             idx_v, coef_v, flat_v, emb_v, first_v, prod_v, score_v, w0_v,
             sem, nc, rows_per_w):
    wid = lax.axis_index("s") * nc + lax.axis_index("c")
    nchunk = rows_per_w // CHUNK
    lane = lax.broadcasted_iota(jnp.int32, (16,), 0)

    pltpu.sync_copy(w0_hbm, w0_v.at[pl.ds(0, 1)])
    w0s = w0_v[pl.ds(0, 16)][0]

    def chunk_body(ci, _):
        rowbase = wid * rows_per_w + ci * CHUNK
        ebase = rowbase * F
        pltpu.sync_copy(idx_hbm.at[pl.ds(ebase, CF)], idx_v)
        pltpu.sync_copy(coef_hbm.at[pl.ds(ebase, CF)], coef_v)

        def slice_body(i, _):
            pos = i * 16 + lane
            vals = idx_v[pl.ds(i * 16, 16)]
            flat_v[i // 8, pl.ds((i % 8) * 16, 16)] = vals + (pos % F) * V
            return 0
        lax.fori_loop(0, NSLICE, slice_body, 0)

        cps = []
        for j in range(F):
            cps.append(pltpu.async_copy(
                second_hbm.at[flat_v.at[j]],
                emb_v.at[pl.ds(j * CHUNK, CHUNK), :], sem))
            cps.append(pltpu.async_copy(
                first_hbm.at[flat_v.at[j]],
                first_v.at[pl.ds(j * CHUNK, CHUNK)], sem))
        for cp in cps:
            cp.wait()

        def prod_body(i, _):
            prod_v[pl.ds(i * 16, 16)] = (first_v[pl.ds(i * 16, 16)] *
                                         coef_v[pl.ds(i * 16, 16)])
            return 0
        lax.fori_loop(0, NSLICE, prod_body, 0)
        prod_v[pl.ds(CF, 16)] = jnp.zeros((16,), jnp.float32)

        def group_body(g, _):
            def row_body(r, svec):
                base = (g * 16 + r) * F
                acc1 = jnp.zeros((16,), jnp.float32)
                acc2 = jnp.zeros((16,), jnp.float32)
                for f in range(F):
                    e = emb_v[base + f, :]
                    cf = plsc.load_gather(
                        coef_v, [jnp.full((16,), base + f, jnp.int32)])
                    t = cf * e
                    acc1 = acc1 + t
                    acc2 = acc2 + t * t
                sec2x = jnp.sum(acc1 * acc1 - acc2)
                v1 = prod_v[pl.ds(base, 16)]
                v2 = prod_v[pl.ds(base + 16, 16)]
                fs = jnp.sum(v1 + jnp.where(lane < (F - 16), v2, 0.0))
                s = w0s + fs + 0.5 * sec2x
                return jnp.where(lane == r, s, svec)
            svec = lax.fori_loop(0, 16, row_body,
                                 jnp.zeros((16,), jnp.float32))
            score_v[pl.ds(g * 16, 16)] = svec
            return 0
        lax.fori_loop(0, CHUNK // 16, group_body, 0)

        pltpu.sync_copy(emb_v, emb_hbm.at[pl.ds(ebase, CF), :])
        pltpu.sync_copy(score_v, score_hbm.at[pl.ds(rowbase, CHUNK)])
        return 0

    lax.fori_loop(0, nchunk, chunk_body, 0)


def kernel(index, coef, w0, first_tables, second_tables):
    info = plsc.get_sparse_core_info()
    nc, ns = info.num_cores, info.num_subcores
    nw = nc * ns
    rows_per_w = B // nw

    idx_flat = index.reshape(B * F)
    coef_flat = coef.reshape(B * F)
    first_flat = first_tables.reshape(F * V)
    second_flat = second_tables.reshape(F * V, D)

    mesh = plsc.VectorSubcoreMesh(core_axis_name="c", subcore_axis_name="s")
    body = functools.partial(_fm_body, nc=nc, rows_per_w=rows_per_w)
    fn = pl.kernel(
        body,
        out_type=(jax.ShapeDtypeStruct((B,), jnp.float32),
                  jax.ShapeDtypeStruct((B * F, D), jnp.float32)),
        mesh=mesh,
        compiler_params=pltpu.CompilerParams(needs_layout_passes=False,
                                             use_tc_tiling_on_sc=False),
        scratch_types=[
            pltpu.VMEM((CF,), jnp.int32),
            pltpu.VMEM((CF,), jnp.float32),
            pltpu.VMEM((F, CHUNK), jnp.int32),
            pltpu.VMEM((CF, D), jnp.float32),
            pltpu.VMEM((CF,), jnp.float32),
            pltpu.VMEM((CF + 16,), jnp.float32),
            pltpu.VMEM((CHUNK,), jnp.float32),
            pltpu.VMEM((16,), jnp.float32),
            pltpu.SemaphoreType.DMA,
        ],
    )
    score, emb = fn(idx_flat, coef_flat, w0, first_flat, second_flat)
    return score, emb.reshape(B, F, D)
